# baseline (speedup 1.0000x reference)
import functools

import jax
import jax.numpy as jnp
from jax import lax
from jax.experimental import pallas as pl
from jax.experimental.pallas import tpu as pltpu
from jax.experimental.pallas import tpu_sc as plsc

N = 10000
D = 128
E = 320000
DE = 16
P = 128
B = 16384
S = 10
S2 = 2

NC = 2
NS = 16
NW = NC * NS
CH = 128

_mesh = plsc.VectorSubcoreMesh(core_axis_name="c", subcore_axis_name="s")
_sc_params = pltpu.CompilerParams(use_tc_tiling_on_sc=False)


def _wid():
    return lax.axis_index("s") * NC + lax.axis_index("c")


def _gather_loop(table_hbm, idx_hbm, out_hbm, idx_v, rows_v, sem, cpw):
    wid = _wid()

    def body(i, carry):
        base = pl.multiple_of((wid * cpw + i) * CH, CH)
        pltpu.sync_copy(idx_hbm.at[pl.ds(base, CH)], idx_v)
        pltpu.async_copy(table_hbm.at[idx_v], rows_v, sem).wait()
        pltpu.sync_copy(rows_v, out_hbm.at[pl.ds(base, CH)])
        return carry

    lax.fori_loop(0, cpw, body, 0)


SPAD = 16


def _sc_gather_eidx(n2e16, ids):
    cpw = B // (NW * CH)

    @functools.partial(
        pl.kernel, mesh=_mesh, compiler_params=_sc_params,
        out_type=jax.ShapeDtypeStruct((B, SPAD), jnp.int32),
        scratch_types=[
            pltpu.VMEM((CH,), jnp.int32),
            pltpu.VMEM((CH, SPAD), jnp.int32),
            pltpu.SemaphoreType.DMA,
        ],
    )
    def k(tbl, idx, out, idx_v, rows_v, sem):
        _gather_loop(tbl, idx, out, idx_v, rows_v, sem, cpw)

    return k(n2e16, ids)


def _sc_gather_edges(edge_node_adj, edge_emb, e_ids):
    M = B * S
    cpw = M // (NW * CH)

    @functools.partial(
        pl.kernel, mesh=_mesh, compiler_params=_sc_params,
        out_type=(
            jax.ShapeDtypeStruct((M, SPAD), jnp.int32),
            jax.ShapeDtypeStruct((M, DE), jnp.float32),
        ),
        scratch_types=[
            pltpu.VMEM((CH,), jnp.int32),
            pltpu.VMEM((CH, SPAD), jnp.int32),
            pltpu.VMEM((CH, DE), jnp.float32),
            pltpu.SemaphoreType.DMA,
        ],
    )
    def k(adj_t, emb_t, idx, adj_o, emb_o, idx_v, adj_v, emb_v, sem):
        wid = _wid()

        def body(i, carry):
            base = pl.multiple_of((wid * cpw + i) * CH, CH)
            pltpu.sync_copy(idx.at[pl.ds(base, CH)], idx_v)
            pltpu.async_copy(adj_t.at[idx_v], adj_v, sem).wait()
            pltpu.async_copy(emb_t.at[idx_v], emb_v, sem).wait()
            pltpu.sync_copy(adj_v, adj_o.at[pl.ds(base, CH)])
            pltpu.sync_copy(emb_v, emb_o.at[pl.ds(base, CH)])
            return carry

        lax.fori_loop(0, cpw, body, 0)

    return k(edge_node_adj, edge_emb, e_ids)


def _sc_gather_feats(feats, n0, n1, ids):
    M = B * S
    cpw = M // (NW * CH)
    cpw0 = B // (NW * CH)

    @functools.partial(
        pl.kernel, mesh=_mesh, compiler_params=_sc_params,
        out_type=(
            jax.ShapeDtypeStruct((M, D), jnp.float32),
            jax.ShapeDtypeStruct((M, D), jnp.float32),
            jax.ShapeDtypeStruct((B, D), jnp.float32),
        ),
        scratch_types=[
            pltpu.VMEM((CH,), jnp.int32),
            pltpu.VMEM((CH, D), jnp.float32),
            pltpu.SemaphoreType.DMA,
        ],
    )
    def k(tbl, n0idx, n1idx, bidx, oute, outo, out0, idx_v, rows_v, sem):
        _gather_loop(tbl, n0idx, oute, idx_v, rows_v, sem, cpw)
        _gather_loop(tbl, n1idx, outo, idx_v, rows_v, sem, cpw)
        _gather_loop(tbl, bidx, out0, idx_v, rows_v, sem, cpw0)

    return k(feats, n0, n1, ids)


TB = 512


def _dg3(x, w):
    return lax.dot_general(x, w, (((2,), (0,)), ((), ())),
                           preferred_element_type=jnp.float32)


def _tc_body(fg0_ref, eg_ref, f2e_ref, f2o_ref, wp_ref, wep_ref, wn0s_ref,
             wn0n_ref, we0s_ref, we0n_ref, wn1s_ref, wn1n_ref, out_ref):
    wp = wp_ref[...]
    wep = wep_ref[...]
    wa = wp @ wn0s_ref[...]
    wb = wep @ wn0n_ref[...]
    wc = wep @ we0s_ref[...]
    wd = wp @ we0n_ref[...]

    fg0 = fg0_ref[...]
    eg = eg_ref[...]
    f2m = (f2e_ref[...] + f2o_ref[...]) * 0.5
    em = jnp.mean(eg, axis=1)

    g0 = jax.nn.relu(fg0 @ wa + em @ wb)
    g1 = jax.nn.relu(_dg3(eg, wc) + _dg3(f2m, wd))
    g1m = jnp.mean(g1, axis=1)
    h0 = jax.nn.relu(g0 @ wn1s_ref[...] + g1m @ wn1n_ref[...])

    out_ref[:, :P] = g0
    out_ref[:, P:] = h0


def _tc_dense(fg0, eg3, f2e, f2o, W_prep, W_edge_prep, Wn0_self, Wn0_neigh,
              We0_self, We0_neigh, Wn1_self, Wn1_neigh):
    grid = (B // TB,)
    wspec = lambda shp: pl.BlockSpec(shp, lambda i: (0,) * len(shp))
    return pl.pallas_call(
        _tc_body,
        grid=grid,
        in_specs=[
            pl.BlockSpec((TB, D), lambda i: (i, 0)),
            pl.BlockSpec((TB, S, DE), lambda i: (i, 0, 0)),
            pl.BlockSpec((TB, S, D), lambda i: (i, 0, 0)),
            pl.BlockSpec((TB, S, D), lambda i: (i, 0, 0)),
            wspec((D, P)),
            wspec((DE, P)),
            wspec((P, P)),
            wspec((P, P)),
            wspec((P, P)),
            wspec((P, P)),
            wspec((P, P)),
            wspec((P, P)),
        ],
        out_specs=pl.BlockSpec((TB, 2 * P), lambda i: (i, 0)),
        out_shape=jax.ShapeDtypeStruct((B, 2 * P), jnp.float32),
    )(fg0, eg3, f2e, f2o, W_prep, W_edge_prep, Wn0_self, Wn0_neigh,
      We0_self, We0_neigh, Wn1_self, Wn1_neigh)


def kernel(ids, feats, edge_emb, node2edge_idx, edge_node_adj,
           W_prep, W_edge_prep, Wn0_self, Wn0_neigh,
           We0_self, We0_neigh, Wn1_self, Wn1_neigh):
    n2e16 = jnp.pad(node2edge_idx, ((0, 0), (0, SPAD - S)))
    adj16 = jnp.pad(edge_node_adj, ((0, 0), (0, SPAD - S2)))
    eidx = _sc_gather_eidx(n2e16, ids)
    e_ids = eidx[:, :S].reshape(-1)
    adj, eg = _sc_gather_edges(adj16, edge_emb, e_ids)
    n0 = adj[:, 0]
    n1 = adj[:, 1]
    f2e, f2o, fg0 = _sc_gather_feats(feats, n0, n1, ids)
    out = _tc_dense(fg0,
                    eg.reshape(B, S, DE),
                    f2e.reshape(B, S, D),
                    f2o.reshape(B, S, D),
                    W_prep, W_edge_prep, Wn0_self, Wn0_neigh,
                    We0_self, We0_neigh, Wn1_self, Wn1_neigh)
    return out[None]


# speedup vs baseline: 1.0009x; 1.0009x over previous
import functools

import jax
import jax.numpy as jnp
from jax import lax
from jax.experimental import pallas as pl
from jax.experimental.pallas import tpu as pltpu
from jax.experimental.pallas import tpu_sc as plsc

N = 10000
D = 128
E = 320000
DE = 16
P = 128
B = 16384
S = 10
S2 = 2

NC = 2
NS = 16
NW = NC * NS
CH = 128

_mesh = plsc.VectorSubcoreMesh(core_axis_name="c", subcore_axis_name="s")
_sc_params = pltpu.CompilerParams(use_tc_tiling_on_sc=False)


def _wid():
    return lax.axis_index("s") * NC + lax.axis_index("c")


def _gather_loop(table_hbm, idx_hbm, out_hbm, idx_v, rows_v, sem, cpw):
    wid = _wid()

    def body(i, carry):
        base = pl.multiple_of((wid * cpw + i) * CH, CH)
        pltpu.sync_copy(idx_hbm.at[pl.ds(base, CH)], idx_v)
        pltpu.async_copy(table_hbm.at[idx_v], rows_v, sem).wait()
        pltpu.sync_copy(rows_v, out_hbm.at[pl.ds(base, CH)])
        return carry

    lax.fori_loop(0, cpw, body, 0)


SPAD = 16


def _sc_gather_eidx(n2e16, ids):
    cpw = B // (NW * CH)

    @functools.partial(
        pl.kernel, mesh=_mesh, compiler_params=_sc_params,
        out_type=jax.ShapeDtypeStruct((B, SPAD), jnp.int32),
        scratch_types=[
            pltpu.VMEM((CH,), jnp.int32),
            pltpu.VMEM((CH, SPAD), jnp.int32),
            pltpu.SemaphoreType.DMA,
        ],
    )
    def k(tbl, idx, out, idx_v, rows_v, sem):
        _gather_loop(tbl, idx, out, idx_v, rows_v, sem, cpw)

    return k(n2e16, ids)


def _sc_gather_edges(edge_node_adj, edge_emb, e_ids):
    M = B * S
    cpw = M // (NW * CH)

    @functools.partial(
        pl.kernel, mesh=_mesh, compiler_params=_sc_params,
        out_type=(
            jax.ShapeDtypeStruct((M, SPAD), jnp.int32),
            jax.ShapeDtypeStruct((M, DE), jnp.float32),
        ),
        scratch_types=[
            pltpu.VMEM((CH,), jnp.int32),
            pltpu.VMEM((CH, SPAD), jnp.int32),
            pltpu.VMEM((CH, DE), jnp.float32),
            pltpu.SemaphoreType.DMA,
        ],
    )
    def k(adj_t, emb_t, idx, adj_o, emb_o, idx_v, adj_v, emb_v, sem):
        wid = _wid()

        def body(i, carry):
            base = pl.multiple_of((wid * cpw + i) * CH, CH)
            pltpu.sync_copy(idx.at[pl.ds(base, CH)], idx_v)
            pltpu.async_copy(adj_t.at[idx_v], adj_v, sem).wait()
            pltpu.async_copy(emb_t.at[idx_v], emb_v, sem).wait()
            pltpu.sync_copy(adj_v, adj_o.at[pl.ds(base, CH)])
            pltpu.sync_copy(emb_v, emb_o.at[pl.ds(base, CH)])
            return carry

        lax.fori_loop(0, cpw, body, 0)

    return k(edge_node_adj, edge_emb, e_ids)


def _sc_gather_feats(feats, n0, n1, ids):
    M = B * S
    cpw = M // (NW * CH)
    cpw0 = B // (NW * CH)

    @functools.partial(
        pl.kernel, mesh=_mesh,
        out_type=(
            jax.ShapeDtypeStruct((M, D), jnp.float32),
            jax.ShapeDtypeStruct((M, D), jnp.float32),
            jax.ShapeDtypeStruct((B, D), jnp.float32),
        ),
        scratch_types=[
            pltpu.VMEM((CH,), jnp.int32),
            pltpu.VMEM((CH, D), jnp.float32),
            pltpu.SemaphoreType.DMA,
        ],
    )
    def k(tbl, n0idx, n1idx, bidx, oute, outo, out0, idx_v, rows_v, sem):
        _gather_loop(tbl, n0idx, oute, idx_v, rows_v, sem, cpw)
        _gather_loop(tbl, n1idx, outo, idx_v, rows_v, sem, cpw)
        _gather_loop(tbl, bidx, out0, idx_v, rows_v, sem, cpw0)

    return k(feats, n0, n1, ids)


TB = 512


def _dg3(x, w):
    return lax.dot_general(x, w, (((2,), (0,)), ((), ())),
                           preferred_element_type=jnp.float32)


def _tc_body(fg0_ref, eg_ref, f2e_ref, f2o_ref, wp_ref, wep_ref, wn0s_ref,
             wn0n_ref, we0s_ref, we0n_ref, wn1s_ref, wn1n_ref, out_ref):
    wp = wp_ref[...]
    wep = wep_ref[...]
    wa = wp @ wn0s_ref[...]
    wb = wep @ wn0n_ref[...]
    wc = wep @ we0s_ref[...]
    wd = (wp @ we0n_ref[...]) * 0.5

    fg0 = fg0_ref[...]
    eg = eg_ref[...]
    f2m = f2e_ref[...] + f2o_ref[...]
    em = jnp.mean(eg, axis=1)

    g0 = jax.nn.relu(fg0 @ wa + em @ wb)
    g1 = jax.nn.relu(_dg3(eg, wc) + _dg3(f2m, wd))
    g1m = jnp.mean(g1, axis=1)
    h0 = jax.nn.relu(g0 @ wn1s_ref[...] + g1m @ wn1n_ref[...])

    out_ref[:, :P] = g0
    out_ref[:, P:] = h0


def _tc_dense(fg0, eg3, f2e, f2o, W_prep, W_edge_prep, Wn0_self, Wn0_neigh,
              We0_self, We0_neigh, Wn1_self, Wn1_neigh):
    grid = (B // TB,)
    wspec = lambda shp: pl.BlockSpec(shp, lambda i: (0,) * len(shp))
    return pl.pallas_call(
        _tc_body,
        grid=grid,
        in_specs=[
            pl.BlockSpec((TB, D), lambda i: (i, 0)),
            pl.BlockSpec((TB, S, DE), lambda i: (i, 0, 0)),
            pl.BlockSpec((TB, S, D), lambda i: (i, 0, 0)),
            pl.BlockSpec((TB, S, D), lambda i: (i, 0, 0)),
            wspec((D, P)),
            wspec((DE, P)),
            wspec((P, P)),
            wspec((P, P)),
            wspec((P, P)),
            wspec((P, P)),
            wspec((P, P)),
            wspec((P, P)),
        ],
        out_specs=pl.BlockSpec((TB, 2 * P), lambda i: (i, 0)),
        out_shape=jax.ShapeDtypeStruct((B, 2 * P), jnp.float32),
    )(fg0, eg3, f2e, f2o, W_prep, W_edge_prep, Wn0_self, Wn0_neigh,
      We0_self, We0_neigh, Wn1_self, Wn1_neigh)


def kernel(ids, feats, edge_emb, node2edge_idx, edge_node_adj,
           W_prep, W_edge_prep, Wn0_self, Wn0_neigh,
           We0_self, We0_neigh, Wn1_self, Wn1_neigh):
    n2e16 = jnp.pad(node2edge_idx, ((0, 0), (0, SPAD - S)))
    adj16 = jnp.pad(edge_node_adj, ((0, 0), (0, SPAD - S2)))
    eidx = _sc_gather_eidx(n2e16, ids)
    e_ids = eidx[:, :S].reshape(-1)
    adj, eg = _sc_gather_edges(adj16, edge_emb, e_ids)
    n0 = adj[:, 0]
    n1 = adj[:, 1]
    f2e, f2o, fg0 = _sc_gather_feats(feats, n0, n1, ids)
    out = _tc_dense(fg0,
                    eg.reshape(B, S, DE),
                    f2e.reshape(B, S, D),
                    f2o.reshape(B, S, D),
                    W_prep, W_edge_prep, Wn0_self, Wn0_neigh,
                    We0_self, We0_neigh, Wn1_self, Wn1_neigh)
    return out[None]


# speedup vs baseline: 1.2521x; 1.2509x over previous
import functools

import jax
import jax.numpy as jnp
from jax import lax
from jax.experimental import pallas as pl
from jax.experimental.pallas import tpu as pltpu
from jax.experimental.pallas import tpu_sc as plsc

N = 10000
D = 128
E = 320000
DE = 16
P = 128
B = 16384
S = 10
S2 = 2

NC = 2
NS = 16
NW = NC * NS
CH = 128

_mesh = plsc.VectorSubcoreMesh(core_axis_name="c", subcore_axis_name="s")
_sc_params = pltpu.CompilerParams(use_tc_tiling_on_sc=False)


def _wid():
    return lax.axis_index("s") * NC + lax.axis_index("c")


def _gather_loop(table_hbm, idx_hbm, out_hbm, idx_v, rows_v, sem, cpw):
    wid = _wid()

    def body(i, carry):
        base = pl.multiple_of((wid * cpw + i) * CH, CH)
        pltpu.sync_copy(idx_hbm.at[pl.ds(base, CH)], idx_v)
        pltpu.async_copy(table_hbm.at[idx_v], rows_v, sem).wait()
        pltpu.sync_copy(rows_v, out_hbm.at[pl.ds(base, CH)])
        return carry

    lax.fori_loop(0, cpw, body, 0)


SPAD = 16


def _sc_gather_eidx(n2e16, ids):
    cpw = B // (NW * CH)

    @functools.partial(
        pl.kernel, mesh=_mesh, compiler_params=_sc_params,
        out_type=jax.ShapeDtypeStruct((B, SPAD), jnp.int32),
        scratch_types=[
            pltpu.VMEM((CH,), jnp.int32),
            pltpu.VMEM((CH, SPAD), jnp.int32),
            pltpu.SemaphoreType.DMA,
        ],
    )
    def k(tbl, idx, out, idx_v, rows_v, sem):
        _gather_loop(tbl, idx, out, idx_v, rows_v, sem, cpw)

    return k(n2e16, ids)


def _sc_gather_edges(edge_node_adj, edge_emb, e_ids):
    M = B * S
    cpw = M // (NW * CH)

    @functools.partial(
        pl.kernel, mesh=_mesh, compiler_params=_sc_params,
        out_type=(
            jax.ShapeDtypeStruct((M, SPAD), jnp.int32),
            jax.ShapeDtypeStruct((M, DE), jnp.float32),
        ),
        scratch_types=[
            pltpu.VMEM((CH,), jnp.int32),
            pltpu.VMEM((CH, SPAD), jnp.int32),
            pltpu.VMEM((CH, DE), jnp.float32),
            pltpu.SemaphoreType.DMA,
        ],
    )
    def k(adj_t, emb_t, idx, adj_o, emb_o, idx_v, adj_v, emb_v, sem):
        wid = _wid()

        def body(i, carry):
            base = pl.multiple_of((wid * cpw + i) * CH, CH)
            pltpu.sync_copy(idx.at[pl.ds(base, CH)], idx_v)
            pltpu.async_copy(adj_t.at[idx_v], adj_v, sem).wait()
            pltpu.async_copy(emb_t.at[idx_v], emb_v, sem).wait()
            pltpu.sync_copy(adj_v, adj_o.at[pl.ds(base, CH)])
            pltpu.sync_copy(emb_v, emb_o.at[pl.ds(base, CH)])
            return carry

        lax.fori_loop(0, cpw, body, 0)

    return k(edge_node_adj, edge_emb, e_ids)


def _sc_gather_feats(feats, n0, n1, ids):
    M = B * S
    cpw = M // (NW * CH)
    cpw0 = B // (NW * CH)

    @functools.partial(
        pl.kernel, mesh=_mesh,
        out_type=(
            jax.ShapeDtypeStruct((M, D), jnp.float32),
            jax.ShapeDtypeStruct((M, D), jnp.float32),
            jax.ShapeDtypeStruct((B, D), jnp.float32),
        ),
        scratch_types=[
            pltpu.VMEM((CH,), jnp.int32),
            pltpu.VMEM((CH, D), jnp.float32),
            pltpu.SemaphoreType.DMA,
        ],
    )
    def k(tbl, n0idx, n1idx, bidx, oute, outo, out0, idx_v, rows_v, sem):
        _gather_loop(tbl, n0idx, oute, idx_v, rows_v, sem, cpw)
        _gather_loop(tbl, n1idx, outo, idx_v, rows_v, sem, cpw)
        _gather_loop(tbl, bidx, out0, idx_v, rows_v, sem, cpw0)

    return k(feats, n0, n1, ids)


TB = 512


def _dg3(x, w):
    return lax.dot_general(x, w, (((2,), (0,)), ((), ())),
                           preferred_element_type=jnp.float32)


def _tc_body(fg0_ref, eg_ref, f2e_ref, f2o_ref, wp_ref, wep_ref, wn0s_ref,
             wn0n_ref, we0s_ref, we0n_ref, wn1s_ref, wn1n_ref, out_ref):
    wp = wp_ref[...]
    wep = wep_ref[...]
    wa = wp @ wn0s_ref[...]
    wb = wep @ wn0n_ref[...]
    wc = wep @ we0s_ref[...]
    wd = (wp @ we0n_ref[...]) * 0.5

    fg0 = fg0_ref[...]
    eg = eg_ref[...]
    f2m = f2e_ref[...] + f2o_ref[...]

    g1 = jax.nn.relu(eg @ wc + f2m @ wd)
    em = jnp.mean(eg.reshape(TB, S, DE), axis=1)
    g0 = jax.nn.relu(fg0 @ wa + em @ wb)
    g1m = jnp.mean(g1.reshape(TB, S, P), axis=1)
    h0 = jax.nn.relu(g0 @ wn1s_ref[...] + g1m @ wn1n_ref[...])

    out_ref[:, :P] = g0
    out_ref[:, P:] = h0


def _tc_dense(fg0, eg, f2e, f2o, W_prep, W_edge_prep, Wn0_self, Wn0_neigh,
              We0_self, We0_neigh, Wn1_self, Wn1_neigh):
    grid = (B // TB,)
    wspec = lambda shp: pl.BlockSpec(shp, lambda i: (0,) * len(shp))
    return pl.pallas_call(
        _tc_body,
        grid=grid,
        in_specs=[
            pl.BlockSpec((TB, D), lambda i: (i, 0)),
            pl.BlockSpec((TB * S, DE), lambda i: (i, 0)),
            pl.BlockSpec((TB * S, D), lambda i: (i, 0)),
            pl.BlockSpec((TB * S, D), lambda i: (i, 0)),
            wspec((D, P)),
            wspec((DE, P)),
            wspec((P, P)),
            wspec((P, P)),
            wspec((P, P)),
            wspec((P, P)),
            wspec((P, P)),
            wspec((P, P)),
        ],
        out_specs=pl.BlockSpec((TB, 2 * P), lambda i: (i, 0)),
        out_shape=jax.ShapeDtypeStruct((B, 2 * P), jnp.float32),
    )(fg0, eg, f2e, f2o, W_prep, W_edge_prep, Wn0_self, Wn0_neigh,
      We0_self, We0_neigh, Wn1_self, Wn1_neigh)


def kernel(ids, feats, edge_emb, node2edge_idx, edge_node_adj,
           W_prep, W_edge_prep, Wn0_self, Wn0_neigh,
           We0_self, We0_neigh, Wn1_self, Wn1_neigh):
    n2e16 = jnp.pad(node2edge_idx, ((0, 0), (0, SPAD - S)))
    adj16 = jnp.pad(edge_node_adj, ((0, 0), (0, SPAD - S2)))
    eidx = _sc_gather_eidx(n2e16, ids)
    e_ids = eidx[:, :S].reshape(-1)
    adj, eg = _sc_gather_edges(adj16, edge_emb, e_ids)
    n0 = adj[:, 0]
    n1 = adj[:, 1]
    f2e, f2o, fg0 = _sc_gather_feats(feats, n0, n1, ids)
    out = _tc_dense(fg0, eg, f2e, f2o,
                    W_prep, W_edge_prep, Wn0_self, Wn0_neigh,
                    We0_self, We0_neigh, Wn1_self, Wn1_neigh)
    return out[None]
